# baseline (speedup 1.0000x reference)
import jax
import jax.numpy as jnp
from jax.experimental import pallas as pl
from jax.experimental.pallas import tpu as pltpu


def _fused_kernel(x_ref, w_ref, adj_ref, out_ref, s_ref):
    i = pl.program_id(0)
    j = pl.program_id(1)
    nj = pl.num_programs(1)
    bj = x_ref.shape[0]
    k = x_ref.shape[1]
    f_out = w_ref.shape[1]

    @pl.when(i == 0)
    def _fill():
        w = w_ref[...]
        for c in range(k):
            s_ref[pl.ds(j * bj, bj), c * f_out:(c + 1) * f_out] = jnp.dot(
                x_ref[:, c, :], w,
                preferred_element_type=jnp.float32).astype(jnp.bfloat16)

    @pl.when(j == nj - 1)
    def _compute():
        a16 = adj_ref[...].astype(jnp.bfloat16)
        acc = jax.lax.dot_general(
            a16, s_ref[...], (((1,), (0,)), ((), ())),
            precision=jax.lax.Precision.DEFAULT,
            preferred_element_type=jnp.float32)
        out_ref[...] = jnp.maximum(acc, 0.0)


def kernel(input, adj, weight):
    n, k, f_in = input.shape
    f_out = weight.shape[1]
    bm = 400
    bj = 2000

    out2d = pl.pallas_call(
        _fused_kernel,
        grid=(n // bm, n // bj),
        in_specs=[
            pl.BlockSpec((bj, k, f_in),
                         lambda i, j: (jnp.where(i == 0, j, 0), 0, 0)),
            pl.BlockSpec((f_in, f_out), lambda i, j: (0, 0)),
            pl.BlockSpec((bm, n), lambda i, j: (i, 0)),
        ],
        out_specs=pl.BlockSpec((bm, k * f_out), lambda i, j: (i, 0)),
        out_shape=jax.ShapeDtypeStruct((n, k * f_out), jnp.float32),
        scratch_shapes=[pltpu.VMEM((n, k * f_out), jnp.bfloat16)],
    )(input, weight, adj)
    return out2d.reshape(n, k, f_out)


# speedup vs baseline: 1.0019x; 1.0019x over previous
import jax
import jax.numpy as jnp
from jax.experimental import pallas as pl
from jax.experimental.pallas import tpu as pltpu


def _fused_kernel(x_ref, w_ref, adj_ref, out_ref, s_ref):
    i = pl.program_id(0)
    j = pl.program_id(1)
    nj = pl.num_programs(1)
    bj = x_ref.shape[0]
    k = x_ref.shape[1]
    f_out = w_ref.shape[1]

    @pl.when(i == 0)
    def _fill():
        w = w_ref[...]
        for c in range(k):
            s_ref[pl.ds(j * bj, bj), c * f_out:(c + 1) * f_out] = jnp.dot(
                x_ref[:, c, :], w, preferred_element_type=jnp.float32)

    @pl.when(j == nj - 1)
    def _compute():
        out_ref[...] = jnp.maximum(
            jnp.dot(adj_ref[...], s_ref[...],
                    preferred_element_type=jnp.float32),
            0.0)


def kernel(input, adj, weight):
    n, k, f_in = input.shape
    f_out = weight.shape[1]
    bm = 400
    bj = 1000

    out2d = pl.pallas_call(
        _fused_kernel,
        grid=(n // bm, n // bj),
        in_specs=[
            pl.BlockSpec((bj, k, f_in),
                         lambda i, j: (jnp.where(i == 0, j, 0), 0, 0)),
            pl.BlockSpec((f_in, f_out), lambda i, j: (0, 0)),
            pl.BlockSpec((bm, n), lambda i, j: (i, 0)),
        ],
        out_specs=pl.BlockSpec((bm, k * f_out), lambda i, j: (i, 0)),
        out_shape=jax.ShapeDtypeStruct((n, k * f_out), jnp.float32),
        scratch_shapes=[pltpu.VMEM((n, k * f_out), jnp.float32)],
    )(input, weight, adj)
    return out2d.reshape(n, k, f_out)
